# baseline (speedup 1.0000x reference)
import functools

import jax
import jax.numpy as jnp
from jax import lax
from jax.experimental import pallas as pl
from jax.experimental.pallas import tpu as pltpu
from jax.experimental.pallas import tpu_sc as plsc

_NUM_CORES = 2
_NUM_SUBCORES = 16
_NUM_WORKERS = _NUM_CORES * _NUM_SUBCORES
_LANES = 16


@functools.lru_cache(maxsize=None)
def _make_gather(batch, hist, vocab, dim):
    bt_total = batch // 128
    bt_per_w = bt_total // _NUM_WORKERS
    n_dt = dim // 8
    mesh = plsc.VectorSubcoreMesh(core_axis_name="c", subcore_axis_name="s")

    scratch = (
        [pltpu.VMEM((128, hist), jnp.int32) for _ in range(2)]
        + [pltpu.VMEM((128,), jnp.int32) for _ in range(2)]
        + [pltpu.VMEM((128, dim), jnp.float32) for _ in range(2)]
        + [pltpu.VMEM((n_dt, 8, 128), jnp.float32) for _ in range(2)]
        + [pltpu.SemaphoreType.DMA]
        + [pltpu.SemaphoreType.DMA] * 2
        + [pltpu.SemaphoreType.DMA] * 2
    )

    @functools.partial(
        pl.kernel,
        mesh=mesh,
        out_type=jax.ShapeDtypeStruct((hist, n_dt, bt_total, 8, 128), jnp.float32),
        scratch_types=scratch,
        compiler_params=pltpu.CompilerParams(use_tc_tiling_on_sc=False, needs_layout_passes=False),
    )
    def gather_kernel(idx_hbm, table_hbm, out_hbm, *bufs):
        idxblk = bufs[0:2]
        idxcol = bufs[2:4]
        gbuf = bufs[4:6]
        tbuf = bufs[6:8]
        isem = bufs[8]
        gsem = bufs[9:11]
        osem = bufs[11:13]

        wid = lax.axis_index("s") * _NUM_CORES + lax.axis_index("c")
        bt0 = wid * bt_per_w

        iota = lax.broadcasted_iota(jnp.int32, (_LANES,), 0)

        def start_idxblk(k, p):
            pltpu.async_copy(
                idx_hbm.at[pl.ds((bt0 + k) * 128, 128)], idxblk[p], isem
            )

        def wait_idxblk(p):
            pltpu.make_async_copy(
                idx_hbm.at[pl.ds(0, 128)], idxblk[p], isem
            ).wait()

        def extract_col(h, kp, up):
            cols = iota * 0 + h
            for j in range(8):
                rows = j * _LANES + iota
                v = plsc.load_gather(idxblk[kp], [rows, cols])
                idxcol[up][pl.ds(j * _LANES, _LANES)] = v

        def start_gather(up):
            pltpu.async_copy(table_hbm.at[idxcol[up]], gbuf[up], gsem[up])

        def wait_gather(up):
            pltpu.make_async_copy(
                table_hbm.at[pl.ds(0, 128)], gbuf[up], gsem[up]
            ).wait()

        def transpose(up):
            g = gbuf[up]
            t = tbuf[up]

            def body_t(ti, carry):
                for i in range(8):
                    cols = iota * 0 + (ti * 8 + i)
                    for j in range(8):
                        rows = j * _LANES + iota
                        v = plsc.load_gather(g, [rows, cols])
                        t[ti, i, pl.ds(j * _LANES, _LANES)] = v
                return carry

            lax.fori_loop(0, n_dt, body_t, 0, unroll=False)

        def start_out(h, k, up):
            pltpu.async_copy(tbuf[up], out_hbm.at[h, :, bt0 + k], osem[up])

        def wait_out(up):
            pltpu.make_async_copy(
                tbuf[up], out_hbm.at[0, :, 0], osem[up]
            ).wait()

        start_idxblk(0, 0)

        for k in range(bt_per_w):
            kp = k % 2
            wait_idxblk(kp)
            if k + 1 < bt_per_w:
                start_idxblk(k + 1, 1 - kp)
            extract_col(0, kp, 0)
            start_gather(0)

            def pair_body(pair, carry, k=k, kp=kp):
                for p in range(2):
                    h = 2 * pair + p
                    nxt = h + 1

                    @pl.when(nxt < hist)
                    def _():
                        extract_col(nxt, kp, 1 - p)
                        start_gather(1 - p)

                    wait_gather(p)
                    if k == 0:
                        @pl.when(h >= 2)
                        def _():
                            wait_out(p)
                    else:
                        wait_out(p)
                    transpose(p)
                    start_out(h, k, p)
                return carry

            lax.fori_loop(0, hist // 2, pair_body, 0, unroll=False)

        wait_out(0)
        wait_out(1)

    return gather_kernel


def kernel(input_ids, table):
    batch, hist = input_ids.shape
    vocab, dim = table.shape
    ids = input_ids.astype(jnp.int32)
    out5 = _make_gather(batch, hist, vocab, dim)(ids, table)
    return out5.transpose(2, 4, 0, 1, 3).reshape(batch, hist, dim)


# speedup vs baseline: 2.9548x; 2.9548x over previous
import functools

import jax
import jax.numpy as jnp
from jax import lax
from jax.experimental import pallas as pl
from jax.experimental.pallas import tpu as pltpu
from jax.experimental.pallas import tpu_sc as plsc

_NUM_CORES = 2
_NUM_SUBCORES = 16
_NUM_WORKERS = _NUM_CORES * _NUM_SUBCORES
_NBUF = 2


@functools.lru_cache(maxsize=None)
def _make_gather(batch, hist, vocab, dim, nr):
    rows_per_w = batch // _NUM_WORKERS
    n_chunks = rows_per_w // nr
    n_groups = n_chunks // _NBUF
    mesh = plsc.VectorSubcoreMesh(core_axis_name="c", subcore_axis_name="s")

    scratch = (
        [pltpu.VMEM((nr, hist), jnp.int32) for _ in range(_NBUF)]
        + [pltpu.VMEM((nr, hist, dim), jnp.float32) for _ in range(_NBUF)]
        + [pltpu.SemaphoreType.DMA for _ in range(3 * _NBUF)]
    )

    @functools.partial(
        pl.kernel,
        mesh=mesh,
        out_type=jax.ShapeDtypeStruct((batch, hist, 2 * dim), jnp.float32),
        scratch_types=scratch,
        compiler_params=pltpu.CompilerParams(use_tc_tiling_on_sc=False),
    )
    def gather_kernel(idx_hbm, table_hbm, out_hbm, *bufs):
        idx_bufs = bufs[0:_NBUF]
        row_bufs = bufs[_NBUF : 2 * _NBUF]
        idx_sems = bufs[2 * _NBUF : 3 * _NBUF]
        g_sems = bufs[3 * _NBUF : 4 * _NBUF]
        out_sems = bufs[4 * _NBUF : 5 * _NBUF]

        wid = lax.axis_index("s") * _NUM_CORES + lax.axis_index("c")
        base = wid * rows_per_w

        def start_idx(ci, b):
            pltpu.async_copy(
                idx_hbm.at[pl.ds(base + ci * nr, nr)], idx_bufs[b], idx_sems[b]
            )

        def wait_idx(b):
            pltpu.make_async_copy(
                idx_hbm.at[pl.ds(0, nr)], idx_bufs[b], idx_sems[b]
            ).wait()

        def start_gathers(b):
            for i in range(nr):
                pltpu.async_copy(
                    table_hbm.at[idx_bufs[b].at[i]], row_bufs[b].at[i], g_sems[b]
                )

        def wait_gathers(b):
            pltpu.make_async_copy(
                out_hbm.at[pl.ds(0, nr), :, pl.ds(0, dim)], row_bufs[b], g_sems[b]
            ).wait()

        def start_out(ci, b):
            pltpu.async_copy(
                row_bufs[b],
                out_hbm.at[pl.ds(base + ci * nr, nr), :, pl.ds(0, dim)],
                out_sems[b],
            )

        def wait_out(b):
            pltpu.make_async_copy(
                row_bufs[b], out_hbm.at[pl.ds(0, nr), :, pl.ds(0, dim)], out_sems[b]
            ).wait()

        for b in range(_NBUF):
            start_idx(b, b)

        def outer(g, carry):
            for b in range(_NBUF):
                ci = g * _NBUF + b
                pb = 1 - b

                wait_idx(b)
                @pl.when(g > 0)
                def _():
                    wait_out(b)

                start_gathers(b)

                def drain_prev():
                    pci = ci - 1
                    wait_gathers(pb)
                    start_out(pci, pb)

                    @pl.when(pci + _NBUF < n_chunks)
                    def _():
                        start_idx(pci + _NBUF, pb)

                if b == 0:
                    @pl.when(g > 0)
                    def _():
                        drain_prev()
                else:
                    drain_prev()

            return carry

        lax.fori_loop(0, n_groups, outer, 0, unroll=False)

        last = n_chunks - 1
        wait_gathers(last % _NBUF)
        start_out(last, last % _NBUF)
        for b in range(_NBUF):
            wait_out(b)

    return gather_kernel


def kernel(input_ids, table):
    batch, hist = input_ids.shape
    vocab, dim = table.shape
    ids = input_ids.astype(jnp.int32)
    out_wide = _make_gather(batch, hist, vocab, dim, 4)(ids, table)
    return out_wide[:, :, :dim]
